# baseline (speedup 1.0000x reference)
import jax
import jax.numpy as jnp
from jax import lax
from jax.experimental import pallas as pl
from jax.experimental.pallas import tpu as pltpu, tpu_sc as plsc

_NUM_FEAT = 64
_NUM_ROWS = 100000
_BATCH = 16384
_F_PER_TILE = _NUM_FEAT // 16
_BCHUNK = 4096
_L = 16


def _gather_feature(idx_v, row_v, out_v, c4):
    @plsc.parallel_loop(0, _BCHUNK, _L, unroll=16)
    def _(k):
        iv = idx_v[pl.ds(c4 * _BCHUNK + k, _L)]
        out_v[pl.ds(k, _L)] = plsc.load_gather(row_v, [iv])


def _do_tables(s, idx_hbm, tables, outs, idx_v, row_v, obufs, sems):
    pltpu.sync_copy(idx_hbm, idx_v)
    handles = [None, None]
    for t, (table, out) in enumerate(zip(tables, outs)):
        for jf in range(_F_PER_TILE):
            f = s * _F_PER_TILE + jf
            pltpu.sync_copy(table.at[f], row_v)
            for c4 in range(_BATCH // _BCHUNK):
                b = c4 % 2
                if handles[b] is not None:
                    handles[b].wait()
                _gather_feature(idx_v, row_v, obufs[b], c4)
                handles[b] = pltpu.async_copy(
                    obufs[b], out.at[f, pl.ds(c4 * _BCHUNK, _BCHUNK)],
                    sems[b])
    for h in handles:
        if h is not None:
            h.wait()


def _body(user_hbm, item_hbm, mfu_t, mfi_t, mlu_t, mli_t,
          out_mfu, out_mfi, out_mlp,
          idx_v, row_v, obuf0, obuf1, sem0, sem1):
    c = lax.axis_index("c")
    s = lax.axis_index("s")
    obufs = (obuf0, obuf1)
    sems = (sem0, sem1)
    @pl.when(c == 0)
    def _():
        _do_tables(s, user_hbm, (mfu_t, mlu_t),
                   (out_mfu, out_mlp.at[pl.ds(0, _NUM_FEAT)]),
                   idx_v, row_v, obufs, sems)

    @pl.when(c == 1)
    def _():
        _do_tables(s, item_hbm, (mfi_t, mli_t),
                   (out_mfi, out_mlp.at[pl.ds(_NUM_FEAT, _NUM_FEAT)]),
                   idx_v, row_v, obufs, sems)


def _run(user_input, item_input, mfu, mfi, mlu, mli):
    mesh = plsc.VectorSubcoreMesh(core_axis_name="c", subcore_axis_name="s")
    fn = pl.kernel(
        _body,
        out_type=(
            jax.ShapeDtypeStruct((_NUM_FEAT, _BATCH), jnp.float32),
            jax.ShapeDtypeStruct((_NUM_FEAT, _BATCH), jnp.float32),
            jax.ShapeDtypeStruct((2 * _NUM_FEAT, _BATCH), jnp.float32),
        ),
        mesh=mesh,
        compiler_params=pltpu.CompilerParams(use_tc_tiling_on_sc=True,
                                             needs_layout_passes=False),
        scratch_types=[
            pltpu.VMEM((_BATCH,), jnp.int32),
            pltpu.VMEM((_NUM_ROWS,), jnp.float32),
            pltpu.VMEM((_BCHUNK,), jnp.float32),
            pltpu.VMEM((_BCHUNK,), jnp.float32),
            pltpu.SemaphoreType.DMA,
            pltpu.SemaphoreType.DMA,
        ],
    )
    mfu_t, mfi_t, mlp_t = fn(user_input, item_input,
                             mfu.T, mfi.T, mlu.T, mli.T)
    return mfu_t.T, mfi_t.T, mlp_t.T


_run_jit = jax.jit(_run)


def kernel(user_input, item_input, mf_user_table, mf_item_table,
           mlp_user_table, mlp_item_table):
    return _run_jit(user_input.astype(jnp.int32),
                    item_input.astype(jnp.int32),
                    mf_user_table, mf_item_table,
                    mlp_user_table, mlp_item_table)


# speedup vs baseline: 1.0429x; 1.0429x over previous
import jax
import jax.numpy as jnp
from jax import lax
from jax.experimental import pallas as pl
from jax.experimental.pallas import tpu as pltpu, tpu_sc as plsc

_NUM_FEAT = 64
_NUM_ROWS = 100000
_BATCH = 16384
_F_PER_TILE = _NUM_FEAT // 16
_BCHUNK = 4096
_L = 16


def _gather_feature(idx_v, row_v, out_v, c4):
    @plsc.parallel_loop(0, _BCHUNK, _L, unroll=8)
    def _(k):
        iv = idx_v[pl.ds(c4 * _BCHUNK + k, _L)]
        out_v[pl.ds(k, _L)] = plsc.load_gather(row_v, [iv])


def _do_tables(s, idx_hbm, tables, outs, idx_v, row_v, obufs, sems):
    ih = pltpu.async_copy(idx_hbm, idx_v, sems[0])
    handles = [None, None]
    first = True
    for t, (table, out) in enumerate(zip(tables, outs)):
        for jf in range(_F_PER_TILE):
            f = s * _F_PER_TILE + jf
            pltpu.sync_copy(table.at[f], row_v)
            if first:
                ih.wait()
                first = False
            for c4 in range(_BATCH // _BCHUNK):
                b = c4 % 2
                if handles[b] is not None:
                    handles[b].wait()
                _gather_feature(idx_v, row_v, obufs[b], c4)
                handles[b] = pltpu.async_copy(
                    obufs[b], out.at[f, pl.ds(c4 * _BCHUNK, _BCHUNK)],
                    sems[b])
    for h in handles:
        if h is not None:
            h.wait()


def _body(user_hbm, item_hbm, mfu_t, mfi_t, mlu_t, mli_t,
          out_mfu, out_mfi, out_mlp,
          idx_v, row_v, obuf0, obuf1, sem0, sem1):
    c = lax.axis_index("c")
    s = lax.axis_index("s")
    obufs = (obuf0, obuf1)
    sems = (sem0, sem1)
    @pl.when(c == 0)
    def _():
        _do_tables(s, user_hbm, (mfu_t, mlu_t),
                   (out_mfu, out_mlp.at[pl.ds(0, _NUM_FEAT)]),
                   idx_v, row_v, obufs, sems)

    @pl.when(c == 1)
    def _():
        _do_tables(s, item_hbm, (mfi_t, mli_t),
                   (out_mfi, out_mlp.at[pl.ds(_NUM_FEAT, _NUM_FEAT)]),
                   idx_v, row_v, obufs, sems)


def _run(user_input, item_input, mfu, mfi, mlu, mli):
    mesh = plsc.VectorSubcoreMesh(core_axis_name="c", subcore_axis_name="s")
    fn = pl.kernel(
        _body,
        out_type=(
            jax.ShapeDtypeStruct((_NUM_FEAT, _BATCH), jnp.float32),
            jax.ShapeDtypeStruct((_NUM_FEAT, _BATCH), jnp.float32),
            jax.ShapeDtypeStruct((2 * _NUM_FEAT, _BATCH), jnp.float32),
        ),
        mesh=mesh,
        compiler_params=pltpu.CompilerParams(use_tc_tiling_on_sc=True,
                                             needs_layout_passes=False),
        scratch_types=[
            pltpu.VMEM((_BATCH,), jnp.int32),
            pltpu.VMEM((_NUM_ROWS,), jnp.float32),
            pltpu.VMEM((_BCHUNK,), jnp.float32),
            pltpu.VMEM((_BCHUNK,), jnp.float32),
            pltpu.SemaphoreType.DMA,
            pltpu.SemaphoreType.DMA,
        ],
    )
    mfu_t, mfi_t, mlp_t = fn(user_input, item_input,
                             mfu.T, mfi.T, mlu.T, mli.T)
    return mfu_t.T, mfi_t.T, mlp_t.T


_run_jit = jax.jit(_run)


def kernel(user_input, item_input, mf_user_table, mf_item_table,
           mlp_user_table, mlp_item_table):
    return _run_jit(user_input.astype(jnp.int32),
                    item_input.astype(jnp.int32),
                    mf_user_table, mf_item_table,
                    mlp_user_table, mlp_item_table)
